# Initial kernel scaffold; baseline (speedup 1.0000x reference)
import jax
import jax.numpy as jnp
from jax.experimental import pallas as pl


def kernel(x, edge_index, edge_weight, Ws, bs):
    raise NotImplementedError("write your pallas kernel here")


# speedup vs baseline: 9.4633x; 9.4633x over previous
import functools

import jax
import jax.numpy as jnp
from jax import lax
from jax.experimental import pallas as pl
from jax.experimental.pallas import tpu as pltpu
from jax.experimental.pallas import tpu_sc as plsc

N_NODES = 10000
N_EDGES = 320000
MP_DIMS = [32, 32, 32, 32, 16, 16, 16, 16, 8, 8, 8, 8]
DELTA_C = 2.0

NC, NS, LANES = 2, 16, 16
NW = NC * NS
M = 128
CH = 80
EPT = CH * M
EPAD = EPT * NW
NPAD = 10240
RPT = NPAD // NS

_f32 = jnp.float32
_i32 = jnp.int32


def _mesh():
    return plsc.VectorSubcoreMesh(
        core_axis_name="c", subcore_axis_name="s", num_cores=NC, num_subcores=NS)


@functools.cache
def _deg_kernel():
    @functools.partial(
        pl.kernel,
        out_type=jax.ShapeDtypeStruct((NC * NPAD,), _f32),
        mesh=_mesh(),
        scratch_types=[
            pltpu.VMEM((CH, M), _i32),
            pltpu.VMEM((CH, M), _f32),
            pltpu.VMEM((RPT,), _f32),
            pltpu.VMEM_SHARED((NPAD,), _f32),
        ],
    )
    def kfn(src_hbm, ew_hbm, out_hbm, src_v, ew_v, buf_v, acc_sh):
        c = lax.axis_index("c")
        s = lax.axis_index("s")
        wid = c * NS + s
        r0 = s * RPT
        zero = jnp.zeros((LANES,), _f32)

        @pl.loop(0, RPT // LANES)
        def _z(i):
            buf_v[pl.ds(i * LANES, LANES)] = zero

        pltpu.sync_copy(buf_v, acc_sh.at[pl.ds(r0, RPT)])
        e0 = wid * CH
        pltpu.sync_copy(src_hbm.at[pl.ds(e0, CH), :], src_v)
        pltpu.sync_copy(ew_hbm.at[pl.ds(e0, CH), :], ew_v)
        plsc.subcore_barrier()

        @pl.loop(0, CH)
        def _chunk(j):
            pltpu.sync_copy(ew_v.at[j], acc_sh.at[src_v.at[j]], add=True)

        plsc.subcore_barrier()
        pltpu.sync_copy(acc_sh.at[pl.ds(r0, RPT)], buf_v)
        pltpu.sync_copy(buf_v, out_hbm.at[pl.ds(c * NPAD + r0, RPT)])

    return kfn


@functools.cache
def _seg_kernel(d):
    @functools.partial(
        pl.kernel,
        out_type=jax.ShapeDtypeStruct((NC, NPAD, d), _f32),
        mesh=_mesh(),
        compiler_params=pltpu.CompilerParams(
            needs_layout_passes=False, use_tc_tiling_on_sc=False),
        scratch_types=[
            pltpu.VMEM((CH, M), _i32),
            pltpu.VMEM((CH, M), _i32),
            pltpu.VMEM((CH, M), _f32),
            pltpu.VMEM((M, d), _f32),
            pltpu.VMEM((RPT, d), _f32),
            pltpu.VMEM_SHARED((NPAD, d), _f32),
            pltpu.SemaphoreType.DMA,
        ],
    )
    def kfn(g_hbm, src_hbm, dst_hbm, ew_hbm, zero_hbm, out_hbm,
            src_v, dst_v, ew_v, rows_v, buf_v, acc_sh, sem):
        c = lax.axis_index("c")
        s = lax.axis_index("s")
        wid = c * NS + s
        r0 = s * RPT
        pltpu.sync_copy(zero_hbm.at[pl.ds(r0, RPT), :], buf_v)
        pltpu.sync_copy(buf_v, acc_sh.at[pl.ds(r0, RPT), :])
        e0 = wid * CH
        pltpu.sync_copy(src_hbm.at[pl.ds(e0, CH), :], src_v)
        pltpu.sync_copy(dst_hbm.at[pl.ds(e0, CH), :], dst_v)
        pltpu.sync_copy(ew_hbm.at[pl.ds(e0, CH), :], ew_v)
        plsc.subcore_barrier()

        iota = lax.iota(_i32, LANES)

        @pl.loop(0, CH)
        def _chunk(j):
            pltpu.async_copy(g_hbm.at[src_v.at[j]], rows_v, sem).wait()
            jf = jnp.full((LANES,), j, _i32)
            if d >= LANES:
                @pl.loop(0, M, unroll=8)
                def _edge(e):
                    cb = plsc.load_gather(ew_v, [jf, jnp.full((LANES,), e, _i32)])
                    for k in range(d // LANES):
                        sl = pl.ds(k * LANES, LANES)
                        rows_v[e, sl] = rows_v[e, sl] * cb
            else:
                per = LANES // d
                rowoff = iota // d
                coloff = iota % d

                @pl.loop(0, M // per, unroll=8)
                def _grp(p):
                    ridx = jnp.full((LANES,), p * per, _i32) + rowoff
                    cb = plsc.load_gather(ew_v, [jf, ridx])
                    vals = plsc.load_gather(rows_v, [ridx, coloff])
                    plsc.store_scatter(rows_v, [ridx, coloff], vals * cb)

            pltpu.sync_copy(rows_v, acc_sh.at[dst_v.at[j]], add=True)

        plsc.subcore_barrier()
        pltpu.sync_copy(acc_sh.at[pl.ds(r0, RPT), :], buf_v)
        pltpu.sync_copy(buf_v, out_hbm.at[c, pl.ds(r0, RPT), :])

    return kfn


def _tc_init_body(x_ref, w_ref, degp_ref, hw_ref, g_ref, dinv_ref):
    deg = degp_ref[0, pl.ds(0, N_NODES), :] + degp_ref[1, pl.ds(0, N_NODES), :]
    dinv = jnp.where(deg > 0.0, lax.rsqrt(jnp.maximum(deg, 1e-30)), 0.0)
    hw = jnp.dot(x_ref[...], w_ref[...], preferred_element_type=_f32)
    hw_ref[...] = hw
    g_ref[...] = DELTA_C * dinv * hw
    dinv_ref[...] = dinv


def _tc_layer_body(accp_ref, hw_ref, dinv_ref, b_ref, w_ref, hw2_ref, g2_ref):
    accsum = accp_ref[0, pl.ds(0, N_NODES), :] + accp_ref[1, pl.ds(0, N_NODES), :]
    dinv = dinv_ref[...]
    h = jnp.tanh(dinv * accsum + (1.0 - DELTA_C) * hw_ref[...] + b_ref[...])
    hw2 = jnp.dot(h, w_ref[...], preferred_element_type=_f32)
    hw2_ref[...] = hw2
    g2_ref[...] = DELTA_C * dinv * hw2


def _tc_final_body(accp_ref, hw_ref, dinv_ref, b_ref,
                   w1_ref, b1_ref, w2_ref, b2_ref, w3_ref, b3_ref, out_ref):
    accsum = accp_ref[0, pl.ds(0, N_NODES), :] + accp_ref[1, pl.ds(0, N_NODES), :]
    h = jnp.tanh(dinv_ref[...] * accsum + (1.0 - DELTA_C) * hw_ref[...] + b_ref[...])
    h = jnp.maximum(jnp.dot(h, w1_ref[...], preferred_element_type=_f32) + b1_ref[...], 0.0)
    h = jnp.maximum(jnp.dot(h, w2_ref[...], preferred_element_type=_f32) + b2_ref[...], 0.0)
    out_ref[...] = jnp.tanh(jnp.dot(h, w3_ref[...], preferred_element_type=_f32) + b3_ref[...])


def _tc_init(x, w0, degp):
    return pl.pallas_call(
        _tc_init_body,
        out_shape=[
            jax.ShapeDtypeStruct((N_NODES, w0.shape[1]), _f32),
            jax.ShapeDtypeStruct((N_NODES, w0.shape[1]), _f32),
            jax.ShapeDtypeStruct((N_NODES, 1), _f32),
        ],
    )(x, w0, degp)


def _tc_layer(accp, hw, dinv, b, w):
    return pl.pallas_call(
        _tc_layer_body,
        out_shape=[
            jax.ShapeDtypeStruct((N_NODES, w.shape[1]), _f32),
            jax.ShapeDtypeStruct((N_NODES, w.shape[1]), _f32),
        ],
    )(accp, hw, dinv, b, w)


def _tc_final(accp, hw, dinv, b, w1, b1, w2, b2, w3, b3):
    return pl.pallas_call(
        _tc_final_body,
        out_shape=jax.ShapeDtypeStruct((N_NODES, 1), _f32),
    )(accp, hw, dinv, b, w1, b1, w2, b2, w3, b3)


def kernel(x, edge_index, edge_weight, Ws, bs):
    src = edge_index[0]
    dst = edge_index[1]
    pad = EPAD - N_EDGES
    srcp = jnp.concatenate([src, jnp.zeros((pad,), _i32)]).reshape(NW * CH, M)
    dstp = jnp.concatenate([dst, jnp.full((pad,), N_NODES, _i32)]).reshape(NW * CH, M)
    ewp = jnp.concatenate([edge_weight, jnp.zeros((pad,), _f32)]).reshape(NW * CH, M)

    zeros = {w: jnp.zeros((NPAD, w), _f32) for w in (32, 16, 8)}

    degp = _deg_kernel()(srcp, ewp).reshape(NC, NPAD, 1)
    hw, g, dinv = _tc_init(x, Ws[0], degp)

    n_mp = len(MP_DIMS)
    for i in range(n_mp):
        d = MP_DIMS[i]
        accp = _seg_kernel(d)(g, srcp, dstp, ewp, zeros[d])
        b = bs[i].reshape(1, d)
        if i < n_mp - 1:
            hw, g = _tc_layer(accp, hw, dinv, b, Ws[i + 1])
        else:
            out = _tc_final(
                accp, hw, dinv, b,
                Ws[n_mp], bs[n_mp].reshape(1, -1),
                Ws[n_mp + 1], bs[n_mp + 1].reshape(1, -1),
                Ws[n_mp + 2], bs[n_mp + 2].reshape(1, -1))
    return out


# speedup vs baseline: 15.4328x; 1.6308x over previous
import functools

import jax
import jax.numpy as jnp
from jax import lax
from jax.experimental import pallas as pl
from jax.experimental.pallas import tpu as pltpu
from jax.experimental.pallas import tpu_sc as plsc

N_NODES = 10000
N_EDGES = 320000
MP_DIMS = [32, 32, 32, 32, 16, 16, 16, 16, 8, 8, 8, 8]
DELTA_C = 2.0

NC, NS, LANES = 2, 16, 16
NW = NC * NS
M = 128
CH = 80
EPT = CH * M
EPAD = EPT * NW
NPAD = 10240
RPT = NPAD // NS

_f32 = jnp.float32
_i32 = jnp.int32


def _mesh():
    return plsc.VectorSubcoreMesh(
        core_axis_name="c", subcore_axis_name="s", num_cores=NC, num_subcores=NS)


@functools.cache
def _deg_kernel():
    @functools.partial(
        pl.kernel,
        out_type=jax.ShapeDtypeStruct((NC * NPAD,), _f32),
        mesh=_mesh(),
        scratch_types=[
            pltpu.VMEM((CH, M), _i32),
            pltpu.VMEM((CH, M), _f32),
            pltpu.VMEM((RPT,), _f32),
            pltpu.VMEM_SHARED((NPAD,), _f32),
        ],
    )
    def kfn(src_hbm, ew_hbm, out_hbm, src_v, ew_v, buf_v, acc_sh):
        c = lax.axis_index("c")
        s = lax.axis_index("s")
        wid = c * NS + s
        r0 = s * RPT
        zero = jnp.zeros((LANES,), _f32)

        @pl.loop(0, RPT // LANES)
        def _z(i):
            buf_v[pl.ds(i * LANES, LANES)] = zero

        pltpu.sync_copy(buf_v, acc_sh.at[pl.ds(r0, RPT)])
        e0 = wid * CH
        pltpu.sync_copy(src_hbm.at[pl.ds(e0, CH), :], src_v)
        pltpu.sync_copy(ew_hbm.at[pl.ds(e0, CH), :], ew_v)
        plsc.subcore_barrier()

        @pl.loop(0, CH)
        def _chunk(j):
            pltpu.sync_copy(ew_v.at[j], acc_sh.at[src_v.at[j]], add=True)

        plsc.subcore_barrier()
        pltpu.sync_copy(acc_sh.at[pl.ds(r0, RPT)], buf_v)
        pltpu.sync_copy(buf_v, out_hbm.at[pl.ds(c * NPAD + r0, RPT)])

    return kfn


NBUF = 4
AHEAD = 2


@functools.cache
def _seg_kernel(d):
    @functools.partial(
        pl.kernel,
        out_type=jax.ShapeDtypeStruct((NC, NPAD, d), _f32),
        mesh=_mesh(),
        compiler_params=pltpu.CompilerParams(
            needs_layout_passes=False, use_tc_tiling_on_sc=False),
        scratch_types=[
            pltpu.VMEM((CH, M), _i32),
            pltpu.VMEM((CH, M), _i32),
            pltpu.VMEM((CH, M), _f32),
            pltpu.VMEM((NBUF, M, d), _f32),
            pltpu.VMEM((RPT, d), _f32),
            pltpu.VMEM_SHARED((NPAD, d), _f32),
            [pltpu.SemaphoreType.DMA] * NBUF,
            [pltpu.SemaphoreType.DMA] * NBUF,
        ],
    )
    def kfn(g_hbm, src_hbm, dst_hbm, ew_hbm, zero_hbm, out_hbm,
            src_v, dst_v, ew_v, rows_v, buf_v, acc_sh, gsems, ssems):
        c = lax.axis_index("c")
        s = lax.axis_index("s")
        wid = c * NS + s
        r0 = s * RPT
        pltpu.sync_copy(zero_hbm.at[pl.ds(r0, RPT), :], buf_v)
        pltpu.sync_copy(buf_v, acc_sh.at[pl.ds(r0, RPT), :])
        e0 = wid * CH
        pltpu.sync_copy(src_hbm.at[pl.ds(e0, CH), :], src_v)
        pltpu.sync_copy(dst_hbm.at[pl.ds(e0, CH), :], dst_v)
        pltpu.sync_copy(ew_hbm.at[pl.ds(e0, CH), :], ew_v)
        plsc.subcore_barrier()

        iota = lax.iota(_i32, LANES)

        def scale(b, j):
            jf = jnp.full((LANES,), j, _i32)
            rb = rows_v.at[b]
            if d >= LANES:
                @pl.loop(0, M, unroll=8)
                def _edge(e):
                    cb = plsc.load_gather(ew_v, [jf, jnp.full((LANES,), e, _i32)])
                    for k in range(d // LANES):
                        sl = pl.ds(k * LANES, LANES)
                        rb[e, sl] = rb[e, sl] * cb
            else:
                per = LANES // d
                rowoff = iota // d
                coloff = iota % d

                @pl.loop(0, M // per, unroll=8)
                def _grp(p):
                    ridx = jnp.full((LANES,), p * per, _i32) + rowoff
                    cb = plsc.load_gather(ew_v, [jf, ridx])
                    vals = plsc.load_gather(rb, [ridx, coloff])
                    plsc.store_scatter(rb, [ridx, coloff], vals * cb)

        def start_gather(j, b):
            pltpu.async_copy(g_hbm.at[src_v.at[j]], rows_v.at[b], gsems[b])

        def wait_gather(b):
            pltpu.make_async_copy(g_hbm.at[src_v.at[0]], rows_v.at[b],
                                  gsems[b]).wait()

        def start_scatter(j, b):
            pltpu.async_copy(rows_v.at[b], acc_sh.at[dst_v.at[j]], ssems[b],
                             add=True)

        def wait_scatter(b):
            pltpu.make_async_copy(rows_v.at[b], acc_sh.at[dst_v.at[0]],
                                  ssems[b]).wait()

        for b in range(AHEAD):
            start_gather(b, b)

        @pl.loop(0, CH // NBUF)
        def _grp(q):
            j0 = q * NBUF
            for b in range(NBUF):
                j = j0 + b
                wait_gather(b)
                scale(b, j)
                start_scatter(j, b)
                jn = j + AHEAD
                bb = (b + AHEAD) % NBUF

                @pl.when(jn < CH)
                def _pre():
                    @pl.when(jn >= NBUF)
                    def _drain():
                        wait_scatter(bb)

                    start_gather(jn, bb)

        for b in range(NBUF - AHEAD, NBUF):
            wait_scatter(b)
        for b in range(0, NBUF - AHEAD):
            wait_scatter(b)

        plsc.subcore_barrier()
        pltpu.sync_copy(acc_sh.at[pl.ds(r0, RPT), :], buf_v)
        pltpu.sync_copy(buf_v, out_hbm.at[c, pl.ds(r0, RPT), :])

    return kfn


def _tc_init_body(x_ref, w_ref, degp_ref, hw_ref, g_ref, dinv_ref):
    deg = degp_ref[0, pl.ds(0, N_NODES), :] + degp_ref[1, pl.ds(0, N_NODES), :]
    dinv = jnp.where(deg > 0.0, lax.rsqrt(jnp.maximum(deg, 1e-30)), 0.0)
    hw = jnp.dot(x_ref[...], w_ref[...], preferred_element_type=_f32)
    hw_ref[...] = hw
    g_ref[...] = DELTA_C * dinv * hw
    dinv_ref[...] = dinv


def _tc_layer_body(accp_ref, hw_ref, dinv_ref, b_ref, w_ref, hw2_ref, g2_ref):
    accsum = accp_ref[0, pl.ds(0, N_NODES), :] + accp_ref[1, pl.ds(0, N_NODES), :]
    dinv = dinv_ref[...]
    h = jnp.tanh(dinv * accsum + (1.0 - DELTA_C) * hw_ref[...] + b_ref[...])
    hw2 = jnp.dot(h, w_ref[...], preferred_element_type=_f32)
    hw2_ref[...] = hw2
    g2_ref[...] = DELTA_C * dinv * hw2


def _tc_final_body(accp_ref, hw_ref, dinv_ref, b_ref,
                   w1_ref, b1_ref, w2_ref, b2_ref, w3_ref, b3_ref, out_ref):
    accsum = accp_ref[0, pl.ds(0, N_NODES), :] + accp_ref[1, pl.ds(0, N_NODES), :]
    h = jnp.tanh(dinv_ref[...] * accsum + (1.0 - DELTA_C) * hw_ref[...] + b_ref[...])
    h = jnp.maximum(jnp.dot(h, w1_ref[...], preferred_element_type=_f32) + b1_ref[...], 0.0)
    h = jnp.maximum(jnp.dot(h, w2_ref[...], preferred_element_type=_f32) + b2_ref[...], 0.0)
    out_ref[...] = jnp.tanh(jnp.dot(h, w3_ref[...], preferred_element_type=_f32) + b3_ref[...])


def _tc_init(x, w0, degp):
    return pl.pallas_call(
        _tc_init_body,
        out_shape=[
            jax.ShapeDtypeStruct((N_NODES, w0.shape[1]), _f32),
            jax.ShapeDtypeStruct((N_NODES, w0.shape[1]), _f32),
            jax.ShapeDtypeStruct((N_NODES, 1), _f32),
        ],
    )(x, w0, degp)


def _tc_layer(accp, hw, dinv, b, w):
    return pl.pallas_call(
        _tc_layer_body,
        out_shape=[
            jax.ShapeDtypeStruct((N_NODES, w.shape[1]), _f32),
            jax.ShapeDtypeStruct((N_NODES, w.shape[1]), _f32),
        ],
    )(accp, hw, dinv, b, w)


def _tc_final(accp, hw, dinv, b, w1, b1, w2, b2, w3, b3):
    return pl.pallas_call(
        _tc_final_body,
        out_shape=jax.ShapeDtypeStruct((N_NODES, 1), _f32),
    )(accp, hw, dinv, b, w1, b1, w2, b2, w3, b3)


def kernel(x, edge_index, edge_weight, Ws, bs):
    src = edge_index[0]
    dst = edge_index[1]
    pad = EPAD - N_EDGES
    srcp = jnp.concatenate([src, jnp.zeros((pad,), _i32)]).reshape(NW * CH, M)
    dstp = jnp.concatenate([dst, jnp.full((pad,), N_NODES, _i32)]).reshape(NW * CH, M)
    ewp = jnp.concatenate([edge_weight, jnp.zeros((pad,), _f32)]).reshape(NW * CH, M)

    zeros = {w: jnp.zeros((NPAD, w), _f32) for w in (32, 16, 8)}

    degp = _deg_kernel()(srcp, ewp).reshape(NC, NPAD, 1)
    hw, g, dinv = _tc_init(x, Ws[0], degp)

    n_mp = len(MP_DIMS)
    for i in range(n_mp):
        d = MP_DIMS[i]
        accp = _seg_kernel(d)(g, srcp, dstp, ewp, zeros[d])
        b = bs[i].reshape(1, d)
        if i < n_mp - 1:
            hw, g = _tc_layer(accp, hw, dinv, b, Ws[i + 1])
        else:
            out = _tc_final(
                accp, hw, dinv, b,
                Ws[n_mp], bs[n_mp].reshape(1, -1),
                Ws[n_mp + 1], bs[n_mp + 1].reshape(1, -1),
                Ws[n_mp + 2], bs[n_mp + 2].reshape(1, -1))
    return out


# speedup vs baseline: 16.0691x; 1.0412x over previous
import functools

import jax
import jax.numpy as jnp
from jax import lax
from jax.experimental import pallas as pl
from jax.experimental.pallas import tpu as pltpu
from jax.experimental.pallas import tpu_sc as plsc

N_NODES = 10000
N_EDGES = 320000
MP_DIMS = [32, 32, 32, 32, 16, 16, 16, 16, 8, 8, 8, 8]
DELTA_C = 2.0

NC, NS, LANES = 2, 16, 16
NW = NC * NS
M = 128
CH = 80
EPT = CH * M
EPAD = EPT * NW
NPAD = 10240
RPT = NPAD // NS

_f32 = jnp.float32
_i32 = jnp.int32


def _mesh():
    return plsc.VectorSubcoreMesh(
        core_axis_name="c", subcore_axis_name="s", num_cores=NC, num_subcores=NS)


@functools.cache
def _deg_kernel():
    @functools.partial(
        pl.kernel,
        out_type=jax.ShapeDtypeStruct((NC * NPAD,), _f32),
        mesh=_mesh(),
        scratch_types=[
            pltpu.VMEM((CH, M), _i32),
            pltpu.VMEM((CH, M), _f32),
            pltpu.VMEM((RPT,), _f32),
            pltpu.VMEM_SHARED((NPAD,), _f32),
        ],
    )
    def kfn(src_hbm, ew_hbm, out_hbm, src_v, ew_v, buf_v, acc_sh):
        c = lax.axis_index("c")
        s = lax.axis_index("s")
        wid = c * NS + s
        r0 = s * RPT
        zero = jnp.zeros((LANES,), _f32)

        @pl.loop(0, RPT // LANES)
        def _z(i):
            buf_v[pl.ds(i * LANES, LANES)] = zero

        pltpu.sync_copy(buf_v, acc_sh.at[pl.ds(r0, RPT)])
        e0 = wid * CH
        pltpu.sync_copy(src_hbm.at[pl.ds(e0, CH), :], src_v)
        pltpu.sync_copy(ew_hbm.at[pl.ds(e0, CH), :], ew_v)
        plsc.subcore_barrier()

        @pl.loop(0, CH)
        def _chunk(j):
            pltpu.sync_copy(ew_v.at[j], acc_sh.at[src_v.at[j]], add=True)

        plsc.subcore_barrier()
        pltpu.sync_copy(acc_sh.at[pl.ds(r0, RPT)], buf_v)
        pltpu.sync_copy(buf_v, out_hbm.at[pl.ds(c * NPAD + r0, RPT)])

    return kfn


NBUF = 8
AHEAD = 6


@functools.cache
def _seg_kernel(d):
    @functools.partial(
        pl.kernel,
        out_type=jax.ShapeDtypeStruct((NC, NPAD, d), _f32),
        mesh=_mesh(),
        compiler_params=pltpu.CompilerParams(
            needs_layout_passes=False, use_tc_tiling_on_sc=False),
        scratch_types=[
            pltpu.VMEM((CH, M), _i32),
            pltpu.VMEM((CH, M), _i32),
            pltpu.VMEM((CH, M), _f32),
            pltpu.VMEM((NBUF, M, d), _f32),
            pltpu.VMEM((RPT, d), _f32),
            pltpu.VMEM_SHARED((NPAD, d), _f32),
            [pltpu.SemaphoreType.DMA] * NBUF,
            [pltpu.SemaphoreType.DMA] * NBUF,
        ],
    )
    def kfn(g_hbm, src_hbm, dst_hbm, ew_hbm, zero_hbm, out_hbm,
            src_v, dst_v, ew_v, rows_v, buf_v, acc_sh, gsems, ssems):
        c = lax.axis_index("c")
        s = lax.axis_index("s")
        wid = c * NS + s
        r0 = s * RPT
        pltpu.sync_copy(zero_hbm.at[pl.ds(r0, RPT), :], buf_v)
        pltpu.sync_copy(buf_v, acc_sh.at[pl.ds(r0, RPT), :])
        e0 = wid * CH
        pltpu.sync_copy(src_hbm.at[pl.ds(e0, CH), :], src_v)
        pltpu.sync_copy(dst_hbm.at[pl.ds(e0, CH), :], dst_v)
        pltpu.sync_copy(ew_hbm.at[pl.ds(e0, CH), :], ew_v)
        plsc.subcore_barrier()

        iota = lax.iota(_i32, LANES)

        def scale(b, j):
            jf = jnp.full((LANES,), j, _i32)
            rb = rows_v.at[b]
            if d >= LANES:
                @pl.loop(0, M, unroll=8)
                def _edge(e):
                    cb = plsc.load_gather(ew_v, [jf, jnp.full((LANES,), e, _i32)])
                    for k in range(d // LANES):
                        sl = pl.ds(k * LANES, LANES)
                        rb[e, sl] = rb[e, sl] * cb
            else:
                per = LANES // d
                rowoff = iota // d
                coloff = iota % d

                @pl.loop(0, M // per, unroll=8)
                def _grp(p):
                    ridx = jnp.full((LANES,), p * per, _i32) + rowoff
                    cb = plsc.load_gather(ew_v, [jf, ridx])
                    vals = plsc.load_gather(rb, [ridx, coloff])
                    plsc.store_scatter(rb, [ridx, coloff], vals * cb)

        def start_gather(j, b):
            pltpu.async_copy(g_hbm.at[src_v.at[j]], rows_v.at[b], gsems[b])

        def wait_gather(b):
            pltpu.make_async_copy(g_hbm.at[src_v.at[0]], rows_v.at[b],
                                  gsems[b]).wait()

        def start_scatter(j, b):
            pltpu.async_copy(rows_v.at[b], acc_sh.at[dst_v.at[j]], ssems[b],
                             add=True)

        def wait_scatter(b):
            pltpu.make_async_copy(rows_v.at[b], acc_sh.at[dst_v.at[0]],
                                  ssems[b]).wait()

        for b in range(AHEAD):
            start_gather(b, b)

        @pl.loop(0, CH // NBUF)
        def _grp(q):
            j0 = q * NBUF
            for b in range(NBUF):
                j = j0 + b
                wait_gather(b)
                scale(b, j)
                start_scatter(j, b)
                jn = j + AHEAD
                bb = (b + AHEAD) % NBUF

                @pl.when(jn < CH)
                def _pre():
                    @pl.when(jn >= NBUF)
                    def _drain():
                        wait_scatter(bb)

                    start_gather(jn, bb)


        for b in range(NBUF):
            wait_scatter(b)

        plsc.subcore_barrier()
        pltpu.sync_copy(acc_sh.at[pl.ds(r0, RPT), :], buf_v)
        pltpu.sync_copy(buf_v, out_hbm.at[c, pl.ds(r0, RPT), :])

    return kfn


def _tc_init_body(x_ref, w_ref, degp_ref, hw_ref, g_ref, dinv_ref):
    deg = degp_ref[0, pl.ds(0, N_NODES), :] + degp_ref[1, pl.ds(0, N_NODES), :]
    dinv = jnp.where(deg > 0.0, lax.rsqrt(jnp.maximum(deg, 1e-30)), 0.0)
    hw = jnp.dot(x_ref[...], w_ref[...], preferred_element_type=_f32)
    hw_ref[...] = hw
    g_ref[...] = DELTA_C * dinv * hw
    dinv_ref[...] = dinv


def _tc_layer_body(accp_ref, hw_ref, dinv_ref, b_ref, w_ref, hw2_ref, g2_ref):
    accsum = accp_ref[0, pl.ds(0, N_NODES), :] + accp_ref[1, pl.ds(0, N_NODES), :]
    dinv = dinv_ref[...]
    h = jnp.tanh(dinv * accsum + (1.0 - DELTA_C) * hw_ref[...] + b_ref[...])
    hw2 = jnp.dot(h, w_ref[...], preferred_element_type=_f32)
    hw2_ref[...] = hw2
    g2_ref[...] = DELTA_C * dinv * hw2


def _tc_final_body(accp_ref, hw_ref, dinv_ref, b_ref,
                   w1_ref, b1_ref, w2_ref, b2_ref, w3_ref, b3_ref, out_ref):
    accsum = accp_ref[0, pl.ds(0, N_NODES), :] + accp_ref[1, pl.ds(0, N_NODES), :]
    h = jnp.tanh(dinv_ref[...] * accsum + (1.0 - DELTA_C) * hw_ref[...] + b_ref[...])
    h = jnp.maximum(jnp.dot(h, w1_ref[...], preferred_element_type=_f32) + b1_ref[...], 0.0)
    h = jnp.maximum(jnp.dot(h, w2_ref[...], preferred_element_type=_f32) + b2_ref[...], 0.0)
    out_ref[...] = jnp.tanh(jnp.dot(h, w3_ref[...], preferred_element_type=_f32) + b3_ref[...])


def _tc_init(x, w0, degp):
    return pl.pallas_call(
        _tc_init_body,
        out_shape=[
            jax.ShapeDtypeStruct((N_NODES, w0.shape[1]), _f32),
            jax.ShapeDtypeStruct((N_NODES, w0.shape[1]), _f32),
            jax.ShapeDtypeStruct((N_NODES, 1), _f32),
        ],
    )(x, w0, degp)


def _tc_layer(accp, hw, dinv, b, w):
    return pl.pallas_call(
        _tc_layer_body,
        out_shape=[
            jax.ShapeDtypeStruct((N_NODES, w.shape[1]), _f32),
            jax.ShapeDtypeStruct((N_NODES, w.shape[1]), _f32),
        ],
    )(accp, hw, dinv, b, w)


def _tc_final(accp, hw, dinv, b, w1, b1, w2, b2, w3, b3):
    return pl.pallas_call(
        _tc_final_body,
        out_shape=jax.ShapeDtypeStruct((N_NODES, 1), _f32),
    )(accp, hw, dinv, b, w1, b1, w2, b2, w3, b3)


def kernel(x, edge_index, edge_weight, Ws, bs):
    src = edge_index[0]
    dst = edge_index[1]
    pad = EPAD - N_EDGES
    srcp = jnp.concatenate([src, jnp.zeros((pad,), _i32)]).reshape(NW * CH, M)
    dstp = jnp.concatenate([dst, jnp.full((pad,), N_NODES, _i32)]).reshape(NW * CH, M)
    ewp = jnp.concatenate([edge_weight, jnp.zeros((pad,), _f32)]).reshape(NW * CH, M)

    zeros = {w: jnp.zeros((NPAD, w), _f32) for w in (32, 16, 8)}

    degp = _deg_kernel()(srcp, ewp).reshape(NC, NPAD, 1)
    hw, g, dinv = _tc_init(x, Ws[0], degp)

    n_mp = len(MP_DIMS)
    for i in range(n_mp):
        d = MP_DIMS[i]
        accp = _seg_kernel(d)(g, srcp, dstp, ewp, zeros[d])
        b = bs[i].reshape(1, d)
        if i < n_mp - 1:
            hw, g = _tc_layer(accp, hw, dinv, b, Ws[i + 1])
        else:
            out = _tc_final(
                accp, hw, dinv, b,
                Ws[n_mp], bs[n_mp].reshape(1, -1),
                Ws[n_mp + 1], bs[n_mp + 1].reshape(1, -1),
                Ws[n_mp + 2], bs[n_mp + 2].reshape(1, -1))
    return out


# speedup vs baseline: 20.2205x; 1.2583x over previous
import functools

import jax
import jax.numpy as jnp
from jax import lax
from jax.experimental import pallas as pl
from jax.experimental.pallas import tpu as pltpu
from jax.experimental.pallas import tpu_sc as plsc

N_NODES = 10000
N_EDGES = 320000
MP_DIMS = [32, 32, 32, 32, 16, 16, 16, 16, 8, 8, 8, 8]
DELTA_C = 2.0

NC, NS, LANES = 2, 16, 16
NW = NC * NS
M = 128
CH = 80
EPT = CH * M
EPAD = EPT * NW
NPAD = 10240
RPT = NPAD // NS

_f32 = jnp.float32
_i32 = jnp.int32


def _mesh():
    return plsc.VectorSubcoreMesh(
        core_axis_name="c", subcore_axis_name="s", num_cores=NC, num_subcores=NS)


@functools.cache
def _deg_kernel():
    @functools.partial(
        pl.kernel,
        out_type=jax.ShapeDtypeStruct((NC * NPAD,), _f32),
        mesh=_mesh(),
        scratch_types=[
            pltpu.VMEM((CH, M), _i32),
            pltpu.VMEM((CH, M), _f32),
            pltpu.VMEM((RPT,), _f32),
            pltpu.VMEM_SHARED((NPAD,), _f32),
        ],
    )
    def kfn(src_hbm, ew_hbm, out_hbm, src_v, ew_v, buf_v, acc_sh):
        c = lax.axis_index("c")
        s = lax.axis_index("s")
        wid = c * NS + s
        r0 = s * RPT
        zero = jnp.zeros((LANES,), _f32)

        @pl.loop(0, RPT // LANES)
        def _z(i):
            buf_v[pl.ds(i * LANES, LANES)] = zero

        pltpu.sync_copy(buf_v, acc_sh.at[pl.ds(r0, RPT)])
        e0 = wid * CH
        pltpu.sync_copy(src_hbm.at[pl.ds(e0, CH), :], src_v)
        pltpu.sync_copy(ew_hbm.at[pl.ds(e0, CH), :], ew_v)
        plsc.subcore_barrier()

        @pl.loop(0, CH)
        def _chunk(j):
            pltpu.sync_copy(ew_v.at[j], acc_sh.at[src_v.at[j]], add=True)

        plsc.subcore_barrier()
        pltpu.sync_copy(acc_sh.at[pl.ds(r0, RPT)], buf_v)
        pltpu.sync_copy(buf_v, out_hbm.at[pl.ds(c * NPAD + r0, RPT)])

    return kfn


NBUF = 8
AHEAD = 6


@functools.cache
def _seg_kernel(d):
    @functools.partial(
        pl.kernel,
        out_type=jax.ShapeDtypeStruct((NC, NPAD, d), _f32),
        mesh=_mesh(),
        compiler_params=pltpu.CompilerParams(
            needs_layout_passes=False, use_tc_tiling_on_sc=False),
        scratch_types=[
            pltpu.VMEM((CH, M), _i32),
            pltpu.VMEM((CH, M), _i32),
            pltpu.VMEM((CH, M), _f32),
            pltpu.VMEM((NBUF, M, d), _f32),
            pltpu.VMEM((RPT, d), _f32),
            pltpu.VMEM_SHARED((NPAD, d), _f32),
            pltpu.VMEM_SHARED((N_NODES, d), _f32),
            [pltpu.SemaphoreType.DMA] * NBUF,
            [pltpu.SemaphoreType.DMA] * NBUF,
        ],
    )
    def kfn(g_hbm, src_hbm, dst_hbm, ew_hbm, zero_hbm, out_hbm,
            src_v, dst_v, ew_v, rows_v, buf_v, acc_sh, g_sh, gsems, ssems):
        c = lax.axis_index("c")
        s = lax.axis_index("s")
        wid = c * NS + s
        r0 = s * RPT
        pltpu.sync_copy(zero_hbm.at[pl.ds(r0, RPT), :], buf_v)
        pltpu.sync_copy(buf_v, acc_sh.at[pl.ds(r0, RPT), :])
        gr0 = s * (N_NODES // NS)
        gbuf = buf_v.at[pl.ds(0, N_NODES // NS), :]
        pltpu.sync_copy(g_hbm.at[pl.ds(gr0, N_NODES // NS), :], gbuf)
        pltpu.sync_copy(gbuf, g_sh.at[pl.ds(gr0, N_NODES // NS), :])
        e0 = wid * CH
        pltpu.sync_copy(src_hbm.at[pl.ds(e0, CH), :], src_v)
        pltpu.sync_copy(dst_hbm.at[pl.ds(e0, CH), :], dst_v)
        pltpu.sync_copy(ew_hbm.at[pl.ds(e0, CH), :], ew_v)
        plsc.subcore_barrier()

        iota = lax.iota(_i32, LANES)

        def scale(b, j):
            jf = jnp.full((LANES,), j, _i32)
            rb = rows_v.at[b]
            if d >= LANES:
                @pl.loop(0, M, unroll=8)
                def _edge(e):
                    cb = plsc.load_gather(ew_v, [jf, jnp.full((LANES,), e, _i32)])
                    for k in range(d // LANES):
                        sl = pl.ds(k * LANES, LANES)
                        rb[e, sl] = rb[e, sl] * cb
            else:
                per = LANES // d
                rowoff = iota // d
                coloff = iota % d

                @pl.loop(0, M // per, unroll=8)
                def _grp(p):
                    ridx = jnp.full((LANES,), p * per, _i32) + rowoff
                    cb = plsc.load_gather(ew_v, [jf, ridx])
                    vals = plsc.load_gather(rb, [ridx, coloff])
                    plsc.store_scatter(rb, [ridx, coloff], vals * cb)

        def start_gather(j, b):
            pltpu.async_copy(g_sh.at[src_v.at[j]], rows_v.at[b], gsems[b])

        def wait_gather(b):
            pltpu.make_async_copy(g_sh.at[src_v.at[0]], rows_v.at[b],
                                  gsems[b]).wait()

        def start_scatter(j, b):
            pltpu.async_copy(rows_v.at[b], acc_sh.at[dst_v.at[j]], ssems[b],
                             add=True)

        def wait_scatter(b):
            pltpu.make_async_copy(rows_v.at[b], acc_sh.at[dst_v.at[0]],
                                  ssems[b]).wait()

        for b in range(AHEAD):
            start_gather(b, b)

        @pl.loop(0, CH // NBUF)
        def _grp(q):
            j0 = q * NBUF
            for b in range(NBUF):
                j = j0 + b
                wait_gather(b)
                scale(b, j)
                start_scatter(j, b)
                jn = j + AHEAD
                bb = (b + AHEAD) % NBUF

                @pl.when(jn < CH)
                def _pre():
                    @pl.when(jn >= NBUF)
                    def _drain():
                        wait_scatter(bb)

                    start_gather(jn, bb)


        for b in range(NBUF):
            wait_scatter(b)

        plsc.subcore_barrier()
        pltpu.sync_copy(acc_sh.at[pl.ds(r0, RPT), :], buf_v)
        pltpu.sync_copy(buf_v, out_hbm.at[c, pl.ds(r0, RPT), :])

    return kfn


def _tc_init_body(x_ref, w_ref, degp_ref, hw_ref, g_ref, dinv_ref):
    deg = degp_ref[0, pl.ds(0, N_NODES), :] + degp_ref[1, pl.ds(0, N_NODES), :]
    dinv = jnp.where(deg > 0.0, lax.rsqrt(jnp.maximum(deg, 1e-30)), 0.0)
    hw = jnp.dot(x_ref[...], w_ref[...], preferred_element_type=_f32)
    hw_ref[...] = hw
    g_ref[...] = DELTA_C * dinv * hw
    dinv_ref[...] = dinv


def _tc_layer_body(accp_ref, hw_ref, dinv_ref, b_ref, w_ref, hw2_ref, g2_ref):
    accsum = accp_ref[0, pl.ds(0, N_NODES), :] + accp_ref[1, pl.ds(0, N_NODES), :]
    dinv = dinv_ref[...]
    h = jnp.tanh(dinv * accsum + (1.0 - DELTA_C) * hw_ref[...] + b_ref[...])
    hw2 = jnp.dot(h, w_ref[...], preferred_element_type=_f32)
    hw2_ref[...] = hw2
    g2_ref[...] = DELTA_C * dinv * hw2


def _tc_final_body(accp_ref, hw_ref, dinv_ref, b_ref,
                   w1_ref, b1_ref, w2_ref, b2_ref, w3_ref, b3_ref, out_ref):
    accsum = accp_ref[0, pl.ds(0, N_NODES), :] + accp_ref[1, pl.ds(0, N_NODES), :]
    h = jnp.tanh(dinv_ref[...] * accsum + (1.0 - DELTA_C) * hw_ref[...] + b_ref[...])
    h = jnp.maximum(jnp.dot(h, w1_ref[...], preferred_element_type=_f32) + b1_ref[...], 0.0)
    h = jnp.maximum(jnp.dot(h, w2_ref[...], preferred_element_type=_f32) + b2_ref[...], 0.0)
    out_ref[...] = jnp.tanh(jnp.dot(h, w3_ref[...], preferred_element_type=_f32) + b3_ref[...])


def _tc_init(x, w0, degp):
    return pl.pallas_call(
        _tc_init_body,
        out_shape=[
            jax.ShapeDtypeStruct((N_NODES, w0.shape[1]), _f32),
            jax.ShapeDtypeStruct((N_NODES, w0.shape[1]), _f32),
            jax.ShapeDtypeStruct((N_NODES, 1), _f32),
        ],
    )(x, w0, degp)


def _tc_layer(accp, hw, dinv, b, w):
    return pl.pallas_call(
        _tc_layer_body,
        out_shape=[
            jax.ShapeDtypeStruct((N_NODES, w.shape[1]), _f32),
            jax.ShapeDtypeStruct((N_NODES, w.shape[1]), _f32),
        ],
    )(accp, hw, dinv, b, w)


def _tc_final(accp, hw, dinv, b, w1, b1, w2, b2, w3, b3):
    return pl.pallas_call(
        _tc_final_body,
        out_shape=jax.ShapeDtypeStruct((N_NODES, 1), _f32),
    )(accp, hw, dinv, b, w1, b1, w2, b2, w3, b3)


def kernel(x, edge_index, edge_weight, Ws, bs):
    src = edge_index[0]
    dst = edge_index[1]
    pad = EPAD - N_EDGES
    srcp = jnp.concatenate([src, jnp.zeros((pad,), _i32)]).reshape(NW * CH, M)
    dstp = jnp.concatenate([dst, jnp.full((pad,), N_NODES, _i32)]).reshape(NW * CH, M)
    ewp = jnp.concatenate([edge_weight, jnp.zeros((pad,), _f32)]).reshape(NW * CH, M)

    zeros = {w: jnp.zeros((NPAD, w), _f32) for w in (32, 16, 8)}

    degp = _deg_kernel()(srcp, ewp).reshape(NC, NPAD, 1)
    hw, g, dinv = _tc_init(x, Ws[0], degp)

    n_mp = len(MP_DIMS)
    for i in range(n_mp):
        d = MP_DIMS[i]
        accp = _seg_kernel(d)(g, srcp, dstp, ewp, zeros[d])
        b = bs[i].reshape(1, d)
        if i < n_mp - 1:
            hw, g = _tc_layer(accp, hw, dinv, b, Ws[i + 1])
        else:
            out = _tc_final(
                accp, hw, dinv, b,
                Ws[n_mp], bs[n_mp].reshape(1, -1),
                Ws[n_mp + 1], bs[n_mp + 1].reshape(1, -1),
                Ws[n_mp + 2], bs[n_mp + 2].reshape(1, -1))
    return out
